# baseline (speedup 1.0000x reference)
import functools

import jax
import jax.numpy as jnp
from jax import lax
from jax.experimental import pallas as pl
from jax.experimental.pallas import tpu as pltpu
from jax.experimental.pallas import tpu_sc as plsc

B = 16
N = 4096
M = 4096
K = 16
L = 16
NBLK = M // L
NW = 32
WPB = NW // B
QPW = N // WPB
P = 4
UA = 4
UB = 4
SD = NBLK


def _knn_body(qt_hbm, kt_hbm, idx_out, dist_out,
              keys_v, q_v, ksq_v, dbuf_v, ci_v, od_v, oi_v):
    cid = lax.axis_index("c")
    sid = lax.axis_index("s")
    wid = sid * 2 + cid
    b = wid // WPB
    h = wid % WPB

    pltpu.sync_copy(kt_hbm.at[b], keys_v)
    pltpu.sync_copy(qt_hbm.at[b], q_v)

    iota = lax.iota(jnp.int32, L)
    zero_i = jnp.zeros((L,), jnp.int32)
    one_i = jnp.full((L,), 1, jnp.int32)
    inf = jnp.float32(jnp.inf)
    inf_v = jnp.full((L,), inf, dtype=jnp.float32)
    fifteen = jnp.full((L,), 15, dtype=jnp.int32)

    gdn = lax.GatherDimensionNumbers(
        offset_dims=(), collapsed_slice_dims=(0,), start_index_map=(0,))

    def gather16(vec, idx):
        return lax.gather(vec, idx[:, None], gdn, (1,),
                          mode=lax.GatherScatterMode.PROMISE_IN_BOUNDS)

    def splat(vec, lane):
        return gather16(vec, jnp.full((L,), lane, dtype=jnp.int32))

    def round_bf16(x):
        bits = plsc.bitcast(x, jnp.int32)
        lsb = lax.shift_right_logical(bits, 16) & 1
        rounded = (bits + 0x7FFF + lsb) & jnp.int32(-65536)
        return plsc.bitcast(rounded, jnp.float32)

    def ksq_body(i, _):
        kx = keys_v[pl.ds(i * L, L)]
        ky = keys_v[pl.ds(M + i * L, L)]
        kz = keys_v[pl.ds(2 * M + i * L, L)]
        ksq_v[pl.ds(i * L, L)] = kx * kx + ky * ky + kz * kz
        keys_v[pl.ds(i * L, L)] = round_bf16(kx)
        keys_v[pl.ds(M + i * L, L)] = round_bf16(ky)
        keys_v[pl.ds(2 * M + i * L, L)] = round_bf16(kz)
        return 0

    lax.fori_loop(0, NBLK, ksq_body, 0)

    stack_base = iota * SD

    def per_pair(p, _):
        q0 = P * p
        g16 = h * QPW + (q0 // L) * L
        lane0 = q0 % L
        qxv = q_v[pl.ds(g16, L)]
        qyv = q_v[pl.ds(N + g16, L)]
        qzv = q_v[pl.ds(2 * N + g16, L)]

        qsq, qx2, qy2, qz2 = [], [], [], []
        for s in range(P):
            qx = splat(qxv, lane0 + s)
            qy = splat(qyv, lane0 + s)
            qz = splat(qzv, lane0 + s)
            qsq.append(qx * qx + qy * qy + qz * qz)
            qx2.append(-2.0 * round_bf16(qx))
            qy2.append(-2.0 * round_bf16(qy))
            qz2.append(-2.0 * round_bf16(qz))

        def pass_a(ib, rs):
            rs = list(rs)
            ks = []
            for j in range(UA):
                i = UA * ib + j
                ks.append((keys_v[pl.ds(i * L, L)],
                           keys_v[pl.ds(M + i * L, L)],
                           keys_v[pl.ds(2 * M + i * L, L)],
                           ksq_v[pl.ds(i * L, L)]))
            for j in range(UA):
                i = UA * ib + j
                kx, ky, kz, ksq = ks[j]
                for s in range(P):
                    dot2 = (qx2[s] * kx + qy2[s] * ky) + qz2[s] * kz
                    d = (dot2 + qsq[s]) + ksq
                    dbuf_v[pl.ds(s * M + i * L, L)] = d
                    rs[s] = jnp.minimum(rs[s], d)
            return tuple(rs)

        rs = lax.fori_loop(0, NBLK // UA, pass_a, (inf_v,) * P)

        ts = []
        for s in range(P):
            rsort, _ = plsc.sort_key_val(rs[s], iota)
            ts.append(gather16(rsort, fifteen))

        def pass_b(ib, poss):
            poss = list(poss)
            dl = []
            for j in range(UB):
                i = UB * ib + j
                for s in range(P):
                    dl.append(dbuf_v[pl.ds(s * M + i * L, L)])
            ml = []
            for j in range(UB):
                for s in range(P):
                    ml.append(dl[j * P + s] <= ts[s])
            for j in range(UB):
                i = UB * ib + j
                for s in range(P):
                    idxv = iota + (i * L + s * M)
                    m = ml[j * P + s]
                    plsc.store_scatter(ci_v, [poss[s]], idxv, mask=m)
                    poss[s] = poss[s] + jnp.where(m, one_i, zero_i)
            return tuple(poss)

        pos0 = tuple(stack_base + s * M for s in range(P))
        poss = lax.fori_loop(0, NBLK // UB, pass_b, pos0)

        cvecs = [poss[s] - pos0[s] for s in range(P)]
        cmax = cvecs[0]
        for s in range(1, P):
            cmax = jnp.maximum(cmax, cvecs[s])
        maxc = jnp.max(cmax)

        def merge(j, c):
            out = []
            for s in range(P):
                T, Ti = c[2 * s], c[2 * s + 1]
                valid = cvecs[s] > j
                gidx = stack_base + (s * M + j)
                ci = plsc.load_gather(ci_v, [gidx], mask=valid)
                cd = plsc.load_gather(dbuf_v, [ci], mask=valid)
                cd = jnp.where(valid, cd, inf)
                cs_, cis_ = plsc.sort_key_val(cd, ci)
                cr = lax.rev(cs_, (0,))
                cir = lax.rev(cis_, (0,))
                keep = T <= cr
                tv = jnp.where(keep, T, cr)
                tiv = jnp.where(keep, Ti, cir)
                t2, ti2 = plsc.sort_key_val(tv, tiv)
                out += [t2, ti2]
            return tuple(out)

        res = lax.fori_loop(0, maxc, merge, (inf_v, zero_i) * P)
        for s in range(P):
            od_v[pl.ds((q0 + s) * K, K)] = res[2 * s]
            oi_v[pl.ds((q0 + s) * K, K)] = res[2 * s + 1] - s * M
        return 0

    lax.fori_loop(0, QPW // P, per_pair, 0)

    pltpu.sync_copy(od_v, dist_out.at[b, pl.ds(h * QPW * K, QPW * K)])
    pltpu.sync_copy(oi_v, idx_out.at[b, pl.ds(h * QPW * K, QPW * K)])


@jax.jit
def _knn(qt, kt):
    mesh = plsc.VectorSubcoreMesh(core_axis_name="c", subcore_axis_name="s")
    f = functools.partial(
        pl.kernel,
        out_type=(
            jax.ShapeDtypeStruct((B, N * K), jnp.int32),
            jax.ShapeDtypeStruct((B, N * K), jnp.float32),
        ),
        mesh=mesh,
        compiler_params=pltpu.CompilerParams(needs_layout_passes=False),
        scratch_types=[
            pltpu.VMEM((3 * M,), jnp.float32),
            pltpu.VMEM((3 * N,), jnp.float32),
            pltpu.VMEM((M,), jnp.float32),
            pltpu.VMEM((P * M,), jnp.float32),
            pltpu.VMEM((P * M,), jnp.int32),
            pltpu.VMEM((QPW * K,), jnp.float32),
            pltpu.VMEM((QPW * K,), jnp.int32),
        ],
    )(_knn_body)
    return f(qt, kt)


def kernel(new_xyz, xyz, k):
    del k
    qt = jnp.transpose(new_xyz, (0, 2, 1)).reshape(B, 3 * N)
    kt = jnp.transpose(xyz, (0, 2, 1)).reshape(B, 3 * M)
    idx, dist = _knn(qt, kt)
    return idx.reshape(B, N, K), dist.reshape(B, N, K)


# speedup vs baseline: 1.0663x; 1.0663x over previous
import functools

import jax
import jax.numpy as jnp
from jax import lax
from jax.experimental import pallas as pl
from jax.experimental.pallas import tpu as pltpu
from jax.experimental.pallas import tpu_sc as plsc

B = 16
N = 4096
M = 4096
K = 16
L = 16
NBLK = M // L
NW = 32
WPB = NW // B
QPW = N // WPB
P = 2
UA = 8
UB = 8
SD = NBLK


def _knn_body(qt_hbm, kt_hbm, idx_out, dist_out,
              keys_v, q_v, ksq_v, dbuf_v, ci_v, od_v, oi_v):
    cid = lax.axis_index("c")
    sid = lax.axis_index("s")
    wid = sid * 2 + cid
    b = wid // WPB
    h = wid % WPB

    pltpu.sync_copy(kt_hbm.at[b], keys_v)
    pltpu.sync_copy(qt_hbm.at[b], q_v)

    iota = lax.iota(jnp.int32, L)
    zero_i = jnp.zeros((L,), jnp.int32)
    one_i = jnp.full((L,), 1, jnp.int32)
    inf = jnp.float32(jnp.inf)
    inf_v = jnp.full((L,), inf, dtype=jnp.float32)
    fifteen = jnp.full((L,), 15, dtype=jnp.int32)

    gdn = lax.GatherDimensionNumbers(
        offset_dims=(), collapsed_slice_dims=(0,), start_index_map=(0,))

    def gather16(vec, idx):
        return lax.gather(vec, idx[:, None], gdn, (1,),
                          mode=lax.GatherScatterMode.PROMISE_IN_BOUNDS)

    def splat(vec, lane):
        return gather16(vec, jnp.full((L,), lane, dtype=jnp.int32))

    def round_bf16(x):
        bits = plsc.bitcast(x, jnp.int32)
        lsb = lax.shift_right_logical(bits, 16) & 1
        rounded = (bits + 0x7FFF + lsb) & jnp.int32(-65536)
        return plsc.bitcast(rounded, jnp.float32)

    def ksq_body(i, _):
        kx = keys_v[pl.ds(i * L, L)]
        ky = keys_v[pl.ds(M + i * L, L)]
        kz = keys_v[pl.ds(2 * M + i * L, L)]
        ksq_v[pl.ds(i * L, L)] = kx * kx + ky * ky + kz * kz
        keys_v[pl.ds(i * L, L)] = round_bf16(kx)
        keys_v[pl.ds(M + i * L, L)] = round_bf16(ky)
        keys_v[pl.ds(2 * M + i * L, L)] = round_bf16(kz)
        return 0

    lax.fori_loop(0, NBLK, ksq_body, 0)

    stack_base = iota * SD

    def per_pair(p, _):
        q0 = P * p
        g16 = h * QPW + (q0 // L) * L
        lane0 = q0 % L
        qxv = q_v[pl.ds(g16, L)]
        qyv = q_v[pl.ds(N + g16, L)]
        qzv = q_v[pl.ds(2 * N + g16, L)]

        qsq, qx2, qy2, qz2 = [], [], [], []
        for s in range(P):
            qx = splat(qxv, lane0 + s)
            qy = splat(qyv, lane0 + s)
            qz = splat(qzv, lane0 + s)
            qsq.append(qx * qx + qy * qy + qz * qz)
            qx2.append(-2.0 * round_bf16(qx))
            qy2.append(-2.0 * round_bf16(qy))
            qz2.append(-2.0 * round_bf16(qz))

        def pass_a(ib, rs):
            rs = list(rs)
            ks = []
            for j in range(UA):
                i = UA * ib + j
                ks.append((keys_v[pl.ds(i * L, L)],
                           keys_v[pl.ds(M + i * L, L)],
                           keys_v[pl.ds(2 * M + i * L, L)],
                           ksq_v[pl.ds(i * L, L)]))
            for j in range(UA):
                i = UA * ib + j
                kx, ky, kz, ksq = ks[j]
                for s in range(P):
                    dot2 = (qx2[s] * kx + qy2[s] * ky) + qz2[s] * kz
                    d = (dot2 + qsq[s]) + ksq
                    dbuf_v[pl.ds(s * M + i * L, L)] = d
                    rs[s] = jnp.minimum(rs[s], d)
            return tuple(rs)

        rs = lax.fori_loop(0, NBLK // UA, pass_a, (inf_v,) * P)

        ts = []
        for s in range(P):
            rsort, _ = plsc.sort_key_val(rs[s], iota)
            ts.append(gather16(rsort, fifteen))

        def pass_b(ib, poss):
            poss = list(poss)
            dl = []
            for j in range(UB):
                i = UB * ib + j
                for s in range(P):
                    dl.append(dbuf_v[pl.ds(s * M + i * L, L)])
            ml = []
            for j in range(UB):
                for s in range(P):
                    ml.append(dl[j * P + s] <= ts[s])
            for j in range(UB):
                i = UB * ib + j
                for s in range(P):
                    idxv = iota + (i * L + s * M)
                    m = ml[j * P + s]
                    plsc.store_scatter(ci_v, [poss[s]], idxv, mask=m)
                    poss[s] = poss[s] + jnp.where(m, one_i, zero_i)
            return tuple(poss)

        pos0 = tuple(stack_base + s * M for s in range(P))
        poss = lax.fori_loop(0, NBLK // UB, pass_b, pos0)

        cvecs = [poss[s] - pos0[s] for s in range(P)]
        maxc = jnp.max(jnp.maximum(cvecs[0], cvecs[1]))

        def merge(j, c):
            out = []
            for s in range(P):
                T, Ti = c[2 * s], c[2 * s + 1]
                valid = cvecs[s] > j
                gidx = stack_base + (s * M + j)
                ci = plsc.load_gather(ci_v, [gidx], mask=valid)
                cd = plsc.load_gather(dbuf_v, [ci], mask=valid)
                cd = jnp.where(valid, cd, inf)
                cs_, cis_ = plsc.sort_key_val(cd, ci)
                cr = lax.rev(cs_, (0,))
                cir = lax.rev(cis_, (0,))
                keep = T <= cr
                tv = jnp.where(keep, T, cr)
                tiv = jnp.where(keep, Ti, cir)
                t2, ti2 = plsc.sort_key_val(tv, tiv)
                out += [t2, ti2]
            return tuple(out)

        res = lax.fori_loop(0, maxc, merge, (inf_v, zero_i) * P)
        for s in range(P):
            od_v[pl.ds((q0 + s) * K, K)] = res[2 * s]
            oi_v[pl.ds((q0 + s) * K, K)] = res[2 * s + 1] - s * M
        return 0

    lax.fori_loop(0, QPW // P, per_pair, 0)

    pltpu.sync_copy(od_v, dist_out.at[b, pl.ds(h * QPW * K, QPW * K)])
    pltpu.sync_copy(oi_v, idx_out.at[b, pl.ds(h * QPW * K, QPW * K)])


@jax.jit
def _knn(qt, kt):
    mesh = plsc.VectorSubcoreMesh(core_axis_name="c", subcore_axis_name="s")
    f = functools.partial(
        pl.kernel,
        out_type=(
            jax.ShapeDtypeStruct((B, N * K), jnp.int32),
            jax.ShapeDtypeStruct((B, N * K), jnp.float32),
        ),
        mesh=mesh,
        compiler_params=pltpu.CompilerParams(needs_layout_passes=False),
        scratch_types=[
            pltpu.VMEM((3 * M,), jnp.float32),
            pltpu.VMEM((3 * N,), jnp.float32),
            pltpu.VMEM((M,), jnp.float32),
            pltpu.VMEM((P * M,), jnp.float32),
            pltpu.VMEM((P * M,), jnp.int32),
            pltpu.VMEM((QPW * K,), jnp.float32),
            pltpu.VMEM((QPW * K,), jnp.int32),
        ],
    )(_knn_body)
    return f(qt, kt)


def kernel(new_xyz, xyz, k):
    del k
    qt = jnp.transpose(new_xyz, (0, 2, 1)).reshape(B, 3 * N)
    kt = jnp.transpose(xyz, (0, 2, 1)).reshape(B, 3 * M)
    idx, dist = _knn(qt, kt)
    return idx.reshape(B, N, K), dist.reshape(B, N, K)
